# baseline (speedup 1.0000x reference)
import functools

import jax
import jax.numpy as jnp
from jax import lax
from jax.experimental import pallas as pl
from jax.experimental.pallas import tpu as pltpu
from jax.experimental.pallas import tpu_sc as plsc

B = 4096
D = 1024
T = 8
C = 1000
CP = 1024
BK1 = 512
G1 = B // BK1
BM = 128
M = B // BM + (T - 1)
XS = M * BM
NC = 2
NS = 16
NW = NC * NS
TPW = B // NW
HH = 32


def _k1_body(x_ref, s_ref, tile_ref, pos_ref, bt_ref, carry_ref, oh_ref,
             rank_ref):
    i = pl.program_id(0)

    @pl.when(i == 0)
    def _():
        carry_ref[...] = jnp.zeros_like(carry_ref)

    @pl.when(i < G1)
    def _():
        sr = s_ref[...]
        sigs = jnp.where(sr > 0.3, 1.0, jnp.where(sr < -0.3, -1.0, 0.0))
        scores = lax.dot_general(x_ref[...], sigs, (((1,), (1,)), ((), ())),
                                 preferred_element_type=jnp.float32)
        idx = jnp.argmax(scores, axis=1).astype(jnp.int32)
        tile_ref[...] = idx[:, None]

        lane = lax.broadcasted_iota(jnp.int32, (BK1, 128), 1)
        oh = (idx[:, None] == lane).astype(jnp.float32)
        r = lax.broadcasted_iota(jnp.int32, (BK1, BK1), 0)
        c = lax.broadcasted_iota(jnp.int32, (BK1, BK1), 1)
        tril = (c < r).astype(jnp.float32)
        csum = lax.dot_general(tril, oh, (((1,), (0,)), ((), ())),
                               preferred_element_type=jnp.float32)
        carry = carry_ref[...]
        rank = jnp.sum((csum + carry) * oh, axis=1)
        oh_ref[pl.ds(i * BK1, BK1), :] = oh
        rank_ref[pl.ds(i * BK1, BK1), :] = rank[:, None]
        carry_ref[...] = carry + jnp.sum(oh, axis=0, keepdims=True)

    @pl.when(i == G1)
    def _():
        counts = carry_ref[...]
        padded = jnp.ceil(counts / BM) * BM
        r2 = lax.broadcasted_iota(jnp.int32, (128, 128), 0)
        c2 = lax.broadcasted_iota(jnp.int32, (128, 128), 1)
        ustrict = (r2 < c2).astype(jnp.float32)
        offs = lax.dot_general(padded, ustrict, (((1,), (0,)), ((), ())),
                               preferred_element_type=jnp.float32)
        eye = (r2 == c2).astype(jnp.float32)
        s_col = lax.dot_general(eye, offs * (1.0 / BM), (((1,), (1,)), ((), ())),
                                preferred_element_type=jnp.float32)
        ind = (c2.astype(jnp.float32) >= s_col).astype(jnp.float32)
        btv = jnp.sum(ind, axis=0, keepdims=True) - 1.0
        btv = jnp.clip(btv, 0.0, float(T - 1))
        lane1 = lax.broadcasted_iota(jnp.int32, (1, 128), 1)
        btv = jnp.where(lane1 == 64, s_col[T, 0], btv)
        bt_ref[...] = btv.astype(jnp.int32)
        seg = lax.dot_general(oh_ref[...], offs, (((1,), (1,)), ((), ())),
                              preferred_element_type=jnp.float32)
        pos_ref[...] = (rank_ref[...] + seg).astype(jnp.int32)


def _k1_call(x, signatures_raw):
    last = G1 - 1
    return pl.pallas_call(
        _k1_body,
        grid=(G1 + 1,),
        in_specs=[
            pl.BlockSpec((BK1, D), lambda i: (jnp.minimum(i, last), 0)),
            pl.BlockSpec((T, D), lambda i: (0, 0)),
        ],
        out_specs=[
            pl.BlockSpec((BK1, 1), lambda i: (jnp.minimum(i, last), 0)),
            pl.BlockSpec((B, 1), lambda i: (0, 0)),
            pl.BlockSpec((1, 128), lambda i: (0, 0)),
        ],
        out_shape=[
            jax.ShapeDtypeStruct((B, 1), jnp.int32),
            jax.ShapeDtypeStruct((B, 1), jnp.int32),
            jax.ShapeDtypeStruct((1, 128), jnp.int32),
        ],
        scratch_shapes=[
            pltpu.VMEM((1, 128), jnp.float32),
            pltpu.VMEM((B, 128), jnp.float32),
            pltpu.VMEM((B, 1), jnp.float32),
        ],
    )(x, signatures_raw)


@functools.cache
def _sc_kernels():
    mesh = plsc.VectorSubcoreMesh(core_axis_name="c", subcore_axis_name="s")
    nch = TPW // HH
    nbuf = 3

    @functools.partial(
        pl.kernel,
        mesh=mesh,
        out_type=jax.ShapeDtypeStruct((XS, D), jnp.float32),
        scratch_types=[
            pltpu.VMEM((nch, HH), jnp.int32),
            pltpu.VMEM((nbuf, HH, D), jnp.float32),
            pltpu.SemaphoreType.DMA,
            pltpu.SemaphoreType.DMA,
        ],
    )
    def k2_scatter(x_hbm, pos_hbm, xs_hbm, pos_v, bufs, in_sem, out_sem):
        wid = lax.axis_index("s") * NC + lax.axis_index("c")
        base = wid * TPW
        pltpu.sync_copy(pos_hbm.at[wid], pos_v)
        ins, outs = {}, {}

        def start_in(h):
            ins[h] = pltpu.async_copy(
                x_hbm.at[pl.ds(base + h * HH, HH)], bufs.at[h % nbuf], in_sem)

        for h in range(nbuf):
            start_in(h)
        for h in range(nch):
            ins[h].wait()
            outs[h] = pltpu.async_copy(
                bufs.at[h % nbuf], xs_hbm.at[pos_v.at[h]], out_sem)
            if h + nbuf < nch:
                outs[h].wait()
                start_in(h + nbuf)
        for h in range(nch):
            if h + nbuf >= nch:
                outs[h].wait()

    @functools.partial(
        pl.kernel,
        mesh=mesh,
        out_type=jax.ShapeDtypeStruct((B, CP), jnp.float32),
        scratch_types=[
            pltpu.VMEM((nch, HH), jnp.int32),
            pltpu.VMEM((nbuf, HH, CP), jnp.float32),
            pltpu.SemaphoreType.DMA,
            pltpu.SemaphoreType.DMA,
        ],
    )
    def k4_gather(ls_hbm, pos_hbm, out_hbm, pos_v, bufs, in_sem, out_sem):
        wid = lax.axis_index("s") * NC + lax.axis_index("c")
        base = wid * TPW
        pltpu.sync_copy(pos_hbm.at[wid], pos_v)
        ins, outs = {}, {}

        def start_in(h):
            ins[h] = pltpu.async_copy(
                ls_hbm.at[pos_v.at[h]], bufs.at[h % nbuf], in_sem)

        for h in range(nbuf):
            start_in(h)
        for h in range(nch):
            ins[h].wait()
            outs[h] = pltpu.async_copy(
                bufs.at[h % nbuf],
                out_hbm.at[pl.ds(base + h * HH, HH)], out_sem)
            if h + nbuf < nch:
                outs[h].wait()
                start_in(h + nbuf)
        for h in range(nch):
            if h + nbuf >= nch:
                outs[h].wait()

    return k2_scatter, k4_gather


def _k3_body(bt_ref, xs_ref, w_ref, b_ref, out_ref):
    m = pl.program_id(0)

    @pl.when(m < bt_ref[64])
    def _():
        t = bt_ref[m]
        wt = w_ref[pl.ds(t, 1)][0]
        bt_b = b_ref[pl.ds(t, 1)][0]
        out_ref[...] = lax.dot_general(
            xs_ref[...].astype(jnp.bfloat16), wt, (((1,), (1,)), ((), ())),
            preferred_element_type=jnp.float32) + bt_b


def _k3_call(bt_flat, xs, W, b3):
    grid_spec = pltpu.PrefetchScalarGridSpec(
        num_scalar_prefetch=1,
        grid=(M,),
        in_specs=[
            pl.BlockSpec((BM, D), lambda m, bt: (m, 0)),
            pl.BlockSpec((T, CP, D), lambda m, bt: (0, 0, 0)),
            pl.BlockSpec((T, 1, CP), lambda m, bt: (0, 0, 0)),
        ],
        out_specs=pl.BlockSpec((BM, CP), lambda m, bt: (m, 0)),
    )
    return pl.pallas_call(
        _k3_body,
        grid_spec=grid_spec,
        out_shape=jax.ShapeDtypeStruct((XS, CP), jnp.float32),
    )(bt_flat, xs, W, b3)


def kernel(x, signatures_raw, W, b):
    k2_scatter, k4_gather = _sc_kernels()
    tile2d, pos2d, bt2d = _k1_call(x, signatures_raw)
    tile_flat = tile2d.reshape(B)
    pos_flat = pos2d.reshape(B)
    bt_flat = bt2d.reshape(128)

    pos4 = pos_flat.reshape(NW, TPW // HH, HH)
    xs = k2_scatter(x, pos4)

    Wbf = jnp.pad(W, ((0, 0), (0, CP - C), (0, 0))).astype(jnp.bfloat16)
    bp = jnp.pad(b, ((0, 0), (0, CP - C))).reshape(T, 1, CP)
    ls = _k3_call(bt_flat, xs, Wbf, bp)

    logits_full = k4_gather(ls, pos4)
    return logits_full[:, :C], tile_flat


# speedup vs baseline: 1.0661x; 1.0661x over previous
import functools

import jax
import jax.numpy as jnp
from jax import lax
from jax.experimental import pallas as pl
from jax.experimental.pallas import tpu as pltpu
from jax.experimental.pallas import tpu_sc as plsc

B = 4096
D = 1024
T = 8
C = 1000
CP = 1024
BK1 = 512
G1 = B // BK1
BM = 128
M = B // BM + (T - 1)
XS = M * BM
NC = 2
NS = 16
NW = NC * NS
TPW = B // NW
HH = 32


def _k1_body(x_ref, s_ref, tile_ref, pos_ref, bt_ref, carry_ref, oh_ref,
             rank_ref):
    i = pl.program_id(0)

    @pl.when(i == 0)
    def _():
        carry_ref[...] = jnp.zeros_like(carry_ref)

    @pl.when(i < G1)
    def _():
        sr = s_ref[...]
        sigs = jnp.where(sr > 0.3, 1.0, jnp.where(sr < -0.3, -1.0, 0.0))
        scores = lax.dot_general(x_ref[...], sigs, (((1,), (1,)), ((), ())),
                                 preferred_element_type=jnp.float32)
        idx = jnp.argmax(scores, axis=1).astype(jnp.int32)
        tile_ref[...] = idx[:, None]

        lane = lax.broadcasted_iota(jnp.int32, (BK1, 128), 1)
        oh = (idx[:, None] == lane).astype(jnp.float32)
        r = lax.broadcasted_iota(jnp.int32, (BK1, BK1), 0)
        c = lax.broadcasted_iota(jnp.int32, (BK1, BK1), 1)
        tril = (c < r).astype(jnp.float32)
        csum = lax.dot_general(tril, oh, (((1,), (0,)), ((), ())),
                               preferred_element_type=jnp.float32)
        carry = carry_ref[...]
        rank = jnp.sum((csum + carry) * oh, axis=1)
        oh_ref[pl.ds(i * BK1, BK1), :] = oh
        rank_ref[pl.ds(i * BK1, BK1), :] = rank[:, None]
        carry_ref[...] = carry + jnp.sum(oh, axis=0, keepdims=True)

    @pl.when(i == G1)
    def _():
        counts = carry_ref[...]
        padded = jnp.ceil(counts / BM) * BM
        r2 = lax.broadcasted_iota(jnp.int32, (128, 128), 0)
        c2 = lax.broadcasted_iota(jnp.int32, (128, 128), 1)
        ustrict = (r2 < c2).astype(jnp.float32)
        offs = lax.dot_general(padded, ustrict, (((1,), (0,)), ((), ())),
                               preferred_element_type=jnp.float32)
        eye = (r2 == c2).astype(jnp.float32)
        s_col = lax.dot_general(eye, offs * (1.0 / BM), (((1,), (1,)), ((), ())),
                                preferred_element_type=jnp.float32)
        ind = (c2.astype(jnp.float32) >= s_col).astype(jnp.float32)
        btv = jnp.sum(ind, axis=0, keepdims=True) - 1.0
        btv = jnp.clip(btv, 0.0, float(T - 1))
        lane1 = lax.broadcasted_iota(jnp.int32, (1, 128), 1)
        btv = jnp.where(lane1 == 64, s_col[T, 0], btv)
        bt_ref[...] = btv.astype(jnp.int32)
        seg = lax.dot_general(oh_ref[...], offs, (((1,), (1,)), ((), ())),
                              preferred_element_type=jnp.float32)
        pos_ref[...] = (rank_ref[...] + seg).astype(jnp.int32)


def _k1_call(x, signatures_raw):
    last = G1 - 1
    return pl.pallas_call(
        _k1_body,
        grid=(G1 + 1,),
        in_specs=[
            pl.BlockSpec((BK1, D), lambda i: (jnp.minimum(i, last), 0)),
            pl.BlockSpec((T, D), lambda i: (0, 0)),
        ],
        out_specs=[
            pl.BlockSpec((BK1, 1), lambda i: (jnp.minimum(i, last), 0)),
            pl.BlockSpec((B, 1), lambda i: (0, 0)),
            pl.BlockSpec((1, 128), lambda i: (0, 0)),
        ],
        out_shape=[
            jax.ShapeDtypeStruct((B, 1), jnp.int32),
            jax.ShapeDtypeStruct((B, 1), jnp.int32),
            jax.ShapeDtypeStruct((1, 128), jnp.int32),
        ],
        scratch_shapes=[
            pltpu.VMEM((1, 128), jnp.float32),
            pltpu.VMEM((B, 128), jnp.float32),
            pltpu.VMEM((B, 1), jnp.float32),
        ],
    )(x, signatures_raw)


@functools.cache
def _sc_kernels():
    mesh = plsc.VectorSubcoreMesh(core_axis_name="c", subcore_axis_name="s")
    nch = TPW // HH
    nbuf = 3

    @functools.partial(
        pl.kernel,
        mesh=mesh,
        out_type=jax.ShapeDtypeStruct((XS, D), jnp.float32),
        scratch_types=[
            pltpu.VMEM((nch, HH), jnp.int32),
            pltpu.VMEM((nbuf, HH, D), jnp.float32),
            pltpu.SemaphoreType.DMA,
            pltpu.SemaphoreType.DMA,
        ],
    )
    def k2_scatter(x_hbm, pos_hbm, xs_hbm, pos_v, bufs, in_sem, out_sem):
        wid = lax.axis_index("s") * NC + lax.axis_index("c")
        base = wid * TPW
        pltpu.sync_copy(pos_hbm.at[wid], pos_v)
        ins, outs = {}, {}

        def start_in(h):
            ins[h] = pltpu.async_copy(
                x_hbm.at[pl.ds(base + h * HH, HH)], bufs.at[h % nbuf], in_sem)

        for h in range(nbuf):
            start_in(h)
        for h in range(nch):
            ins[h].wait()
            outs[h] = pltpu.async_copy(
                bufs.at[h % nbuf], xs_hbm.at[pos_v.at[h]], out_sem)
            if h + nbuf < nch:
                outs[h].wait()
                start_in(h + nbuf)
        for h in range(nch):
            if h + nbuf >= nch:
                outs[h].wait()

    @functools.partial(
        pl.kernel,
        mesh=mesh,
        out_type=jax.ShapeDtypeStruct((B, CP), jnp.float32),
        scratch_types=[
            pltpu.VMEM((nch, HH), jnp.int32),
            pltpu.VMEM((nbuf, HH, CP), jnp.float32),
            pltpu.SemaphoreType.DMA,
            pltpu.SemaphoreType.DMA,
        ],
    )
    def k4_gather(ls_hbm, pos_hbm, out_hbm, pos_v, bufs, in_sem, out_sem):
        wid = lax.axis_index("s") * NC + lax.axis_index("c")
        base = wid * TPW
        pltpu.sync_copy(pos_hbm.at[wid], pos_v)
        ins, outs = {}, {}

        def start_in(h):
            ins[h] = pltpu.async_copy(
                ls_hbm.at[pos_v.at[h]], bufs.at[h % nbuf], in_sem)

        for h in range(nbuf):
            start_in(h)
        for h in range(nch):
            ins[h].wait()
            outs[h] = pltpu.async_copy(
                bufs.at[h % nbuf],
                out_hbm.at[pl.ds(base + h * HH, HH)], out_sem)
            if h + nbuf < nch:
                outs[h].wait()
                start_in(h + nbuf)
        for h in range(nch):
            if h + nbuf >= nch:
                outs[h].wait()

    return k2_scatter, k4_gather


def _k3_body(bt_ref, xs_ref, w_ref, b_ref, out_ref, wbf_ref):
    m = pl.program_id(0)

    @pl.when(m == 0)
    def _():
        wbf_ref[pl.ds(C, CP - C)] = jnp.zeros((CP - C, D), jnp.bfloat16)

    @pl.when(m < bt_ref[64])
    def _():
        tprev = bt_ref[jnp.maximum(m - 1, 0)]

        @pl.when(jnp.logical_or(m == 0, bt_ref[m] != tprev))
        def _():
            wbf_ref[pl.ds(0, C)] = w_ref[0].astype(jnp.bfloat16)

        out_ref[...] = lax.dot_general(
            xs_ref[...].astype(jnp.bfloat16), wbf_ref[...],
            (((1,), (1,)), ((), ())),
            preferred_element_type=jnp.float32) + b_ref[0]


def _k3_call(bt_flat, xs, W, bp):
    grid_spec = pltpu.PrefetchScalarGridSpec(
        num_scalar_prefetch=1,
        grid=(M,),
        in_specs=[
            pl.BlockSpec((BM, D), lambda m, bt: (m, 0)),
            pl.BlockSpec((1, C, D), lambda m, bt: (bt[m], 0, 0)),
            pl.BlockSpec((1, 1, CP), lambda m, bt: (bt[m], 0, 0)),
        ],
        out_specs=pl.BlockSpec((BM, CP), lambda m, bt: (m, 0)),
        scratch_shapes=[pltpu.VMEM((CP, D), jnp.bfloat16)],
    )
    return pl.pallas_call(
        _k3_body,
        grid_spec=grid_spec,
        out_shape=jax.ShapeDtypeStruct((XS, CP), jnp.float32),
    )(bt_flat, xs, W, bp)


def kernel(x, signatures_raw, W, b):
    k2_scatter, k4_gather = _sc_kernels()
    tile2d, pos2d, bt2d = _k1_call(x, signatures_raw)
    tile_flat = tile2d.reshape(B)
    pos_flat = pos2d.reshape(B)
    bt_flat = bt2d.reshape(128)

    pos4 = pos_flat.reshape(NW, TPW // HH, HH)
    xs = k2_scatter(x, pos4)

    bp = jnp.pad(b, ((0, 0), (0, CP - C))).reshape(T, 1, CP)
    ls = _k3_call(bt_flat, xs, W, bp)

    logits_full = k4_gather(ls, pos4)
    return logits_full[:, :C], tile_flat


# speedup vs baseline: 1.1143x; 1.0452x over previous
import functools

import jax
import jax.numpy as jnp
from jax import lax
from jax.experimental import pallas as pl
from jax.experimental.pallas import tpu as pltpu
from jax.experimental.pallas import tpu_sc as plsc

B = 4096
D = 1024
T = 8
C = 1000
CP = 1024
BK1 = 512
G1 = B // BK1
BM = 128
M = B // BM + (T - 1)
XS = M * BM
DP = D // 2
NC = 2
NS = 16
NW = NC * NS
TPW = B // NW
HH = 32


def _k1_body(x_ref, s_ref, tile_ref, xb_ref, pos_ref, bt_ref, carry_ref,
             oh_ref, rank_ref):
    i = pl.program_id(0)

    @pl.when(i == 0)
    def _():
        carry_ref[...] = jnp.zeros_like(carry_ref)

    @pl.when(i < G1)
    def _():
        xv = x_ref[...]
        u = lax.bitcast_convert_type(
            xv.astype(jnp.bfloat16), jnp.uint16).astype(jnp.int32)
        xb_ref[...] = u[:, :DP] | (u[:, DP:] << 16)
        sr = s_ref[...]
        sigs = jnp.where(sr > 0.3, 1.0, jnp.where(sr < -0.3, -1.0, 0.0))
        scores = lax.dot_general(xv, sigs, (((1,), (1,)), ((), ())),
                                 preferred_element_type=jnp.float32)
        idx = jnp.argmax(scores, axis=1).astype(jnp.int32)
        tile_ref[...] = idx[:, None]

        lane = lax.broadcasted_iota(jnp.int32, (BK1, 128), 1)
        oh = (idx[:, None] == lane).astype(jnp.float32)
        r = lax.broadcasted_iota(jnp.int32, (BK1, BK1), 0)
        c = lax.broadcasted_iota(jnp.int32, (BK1, BK1), 1)
        tril = (c < r).astype(jnp.float32)
        csum = lax.dot_general(tril, oh, (((1,), (0,)), ((), ())),
                               preferred_element_type=jnp.float32)
        carry = carry_ref[...]
        rank = jnp.sum((csum + carry) * oh, axis=1)
        oh_ref[pl.ds(i * BK1, BK1), :] = oh
        rank_ref[pl.ds(i * BK1, BK1), :] = rank[:, None]
        carry_ref[...] = carry + jnp.sum(oh, axis=0, keepdims=True)

    @pl.when(i == G1)
    def _():
        counts = carry_ref[...]
        padded = jnp.ceil(counts / BM) * BM
        r2 = lax.broadcasted_iota(jnp.int32, (128, 128), 0)
        c2 = lax.broadcasted_iota(jnp.int32, (128, 128), 1)
        ustrict = (r2 < c2).astype(jnp.float32)
        offs = lax.dot_general(padded, ustrict, (((1,), (0,)), ((), ())),
                               preferred_element_type=jnp.float32)
        eye = (r2 == c2).astype(jnp.float32)
        s_col = lax.dot_general(eye, offs * (1.0 / BM), (((1,), (1,)), ((), ())),
                                preferred_element_type=jnp.float32)
        ind = (c2.astype(jnp.float32) >= s_col).astype(jnp.float32)
        btv = jnp.sum(ind, axis=0, keepdims=True) - 1.0
        btv = jnp.clip(btv, 0.0, float(T - 1))
        lane1 = lax.broadcasted_iota(jnp.int32, (1, 128), 1)
        btv = jnp.where(lane1 == 64, s_col[T, 0], btv)
        bt_ref[...] = btv.astype(jnp.int32)
        seg = lax.dot_general(oh_ref[...], offs, (((1,), (1,)), ((), ())),
                              preferred_element_type=jnp.float32)
        pos_ref[...] = (rank_ref[...] + seg).astype(jnp.int32)


def _k1_call(x, signatures_raw):
    last = G1 - 1
    return pl.pallas_call(
        _k1_body,
        grid=(G1 + 1,),
        in_specs=[
            pl.BlockSpec((BK1, D), lambda i: (jnp.minimum(i, last), 0)),
            pl.BlockSpec((T, D), lambda i: (0, 0)),
        ],
        out_specs=[
            pl.BlockSpec((BK1, 1), lambda i: (jnp.minimum(i, last), 0)),
            pl.BlockSpec((BK1, DP), lambda i: (jnp.minimum(i, last), 0)),
            pl.BlockSpec((B, 1), lambda i: (0, 0)),
            pl.BlockSpec((1, 128), lambda i: (0, 0)),
        ],
        out_shape=[
            jax.ShapeDtypeStruct((B, 1), jnp.int32),
            jax.ShapeDtypeStruct((B, DP), jnp.int32),
            jax.ShapeDtypeStruct((B, 1), jnp.int32),
            jax.ShapeDtypeStruct((1, 128), jnp.int32),
        ],
        scratch_shapes=[
            pltpu.VMEM((1, 128), jnp.float32),
            pltpu.VMEM((B, 128), jnp.float32),
            pltpu.VMEM((B, 1), jnp.float32),
        ],
    )(x, signatures_raw)


@functools.cache
def _sc_kernels():
    mesh = plsc.VectorSubcoreMesh(core_axis_name="c", subcore_axis_name="s")
    nch = TPW // HH
    nbuf = 3

    @functools.partial(
        pl.kernel,
        mesh=mesh,
        out_type=jax.ShapeDtypeStruct((XS, DP), jnp.int32),
        scratch_types=[
            pltpu.VMEM((nch, HH), jnp.int32),
            pltpu.VMEM((nbuf, HH, DP), jnp.int32),
            pltpu.SemaphoreType.DMA,
            pltpu.SemaphoreType.DMA,
        ],
    )
    def k2_scatter(x_hbm, pos_hbm, xs_hbm, pos_v, bufs, in_sem, out_sem):
        wid = lax.axis_index("s") * NC + lax.axis_index("c")
        base = wid * TPW
        pltpu.sync_copy(pos_hbm.at[wid], pos_v)
        ins, outs = {}, {}

        def start_in(h):
            ins[h] = pltpu.async_copy(
                x_hbm.at[pl.ds(base + h * HH, HH)], bufs.at[h % nbuf], in_sem)

        for h in range(nbuf):
            start_in(h)
        for h in range(nch):
            ins[h].wait()
            outs[h] = pltpu.async_copy(
                bufs.at[h % nbuf], xs_hbm.at[pos_v.at[h]], out_sem)
            if h + nbuf < nch:
                outs[h].wait()
                start_in(h + nbuf)
        for h in range(nch):
            if h + nbuf >= nch:
                outs[h].wait()

    @functools.partial(
        pl.kernel,
        mesh=mesh,
        out_type=jax.ShapeDtypeStruct((B, CP), jnp.float32),
        scratch_types=[
            pltpu.VMEM((nch, HH), jnp.int32),
            pltpu.VMEM((nbuf, HH, CP), jnp.float32),
            pltpu.SemaphoreType.DMA,
            pltpu.SemaphoreType.DMA,
        ],
    )
    def k4_gather(ls_hbm, pos_hbm, out_hbm, pos_v, bufs, in_sem, out_sem):
        wid = lax.axis_index("s") * NC + lax.axis_index("c")
        base = wid * TPW
        pltpu.sync_copy(pos_hbm.at[wid], pos_v)
        ins, outs = {}, {}

        def start_in(h):
            ins[h] = pltpu.async_copy(
                ls_hbm.at[pos_v.at[h]], bufs.at[h % nbuf], in_sem)

        for h in range(nbuf):
            start_in(h)
        for h in range(nch):
            ins[h].wait()
            outs[h] = pltpu.async_copy(
                bufs.at[h % nbuf],
                out_hbm.at[pl.ds(base + h * HH, HH)], out_sem)
            if h + nbuf < nch:
                outs[h].wait()
                start_in(h + nbuf)
        for h in range(nch):
            if h + nbuf >= nch:
                outs[h].wait()

    return k2_scatter, k4_gather


def _k3_body(bt_ref, xs_ref, w_ref, b_ref, out_ref, wbf_ref):
    m = pl.program_id(0)

    @pl.when(m == 0)
    def _():
        wbf_ref[pl.ds(C, CP - C)] = jnp.zeros((CP - C, D), jnp.bfloat16)

    @pl.when(m < bt_ref[64])
    def _():
        tprev = bt_ref[jnp.maximum(m - 1, 0)]

        @pl.when(jnp.logical_or(m == 0, bt_ref[m] != tprev))
        def _():
            wbf_ref[pl.ds(0, C)] = w_ref[0].astype(jnp.bfloat16)

        v = xs_ref[...]
        lo = lax.bitcast_convert_type(v << 16, jnp.float32)
        hi = lax.bitcast_convert_type(v & jnp.int32(-65536), jnp.float32)
        xb = jnp.concatenate([lo, hi], axis=1).astype(jnp.bfloat16)
        out_ref[...] = lax.dot_general(
            xb, wbf_ref[...], (((1,), (1,)), ((), ())),
            preferred_element_type=jnp.float32) + b_ref[0]


def _k3_call(bt_flat, xs, W, bp):
    grid_spec = pltpu.PrefetchScalarGridSpec(
        num_scalar_prefetch=1,
        grid=(M,),
        in_specs=[
            pl.BlockSpec((BM, DP), lambda m, bt: (m, 0)),
            pl.BlockSpec((1, C, D), lambda m, bt: (bt[m], 0, 0)),
            pl.BlockSpec((1, 1, CP), lambda m, bt: (bt[m], 0, 0)),
        ],
        out_specs=pl.BlockSpec((BM, CP), lambda m, bt: (m, 0)),
        scratch_shapes=[pltpu.VMEM((CP, D), jnp.bfloat16)],
    )
    return pl.pallas_call(
        _k3_body,
        grid_spec=grid_spec,
        out_shape=jax.ShapeDtypeStruct((XS, CP), jnp.float32),
    )(bt_flat, xs, W, bp)


def kernel(x, signatures_raw, W, b):
    k2_scatter, k4_gather = _sc_kernels()
    tile2d, xb, pos2d, bt2d = _k1_call(x, signatures_raw)
    tile_flat = tile2d.reshape(B)
    pos_flat = pos2d.reshape(B)
    bt_flat = bt2d.reshape(128)

    pos4 = pos_flat.reshape(NW, TPW // HH, HH)
    xs = k2_scatter(xb, pos4)

    bp = jnp.pad(b, ((0, 0), (0, CP - C))).reshape(T, 1, CP)
    ls = _k3_call(bt_flat, xs, W, bp)

    logits_full = k4_gather(ls, pos4)
    return logits_full[:, :C], tile_flat


# speedup vs baseline: 1.1160x; 1.0015x over previous
import functools

import jax
import jax.numpy as jnp
from jax import lax
from jax.experimental import pallas as pl
from jax.experimental.pallas import tpu as pltpu
from jax.experimental.pallas import tpu_sc as plsc

B = 4096
D = 1024
T = 8
C = 1000
CP = 1024
BK1 = 512
G1 = B // BK1
BM = 128
M = B // BM + (T - 1)
XS = M * BM
DP = D // 2
NC = 2
NS = 16
NW = NC * NS
TPW = B // NW
HH = 32


def _k1_body(x_ref, s_ref, tile_ref, xb_ref, pos_ref, bt_ref, carry_ref,
             oh_ref, rank_ref):
    i = pl.program_id(0)

    @pl.when(i == 0)
    def _():
        carry_ref[...] = jnp.zeros_like(carry_ref)

    @pl.when(i < G1)
    def _():
        xv = x_ref[...]
        u = lax.bitcast_convert_type(
            xv.astype(jnp.bfloat16), jnp.uint16).astype(jnp.int32)
        xb_ref[...] = u[:, :DP] | (u[:, DP:] << 16)
        sr = s_ref[...]
        sigs = jnp.where(sr > 0.3, 1.0, jnp.where(sr < -0.3, -1.0, 0.0))
        scores = lax.dot_general(xv, sigs, (((1,), (1,)), ((), ())),
                                 preferred_element_type=jnp.float32)
        idx = jnp.argmax(scores, axis=1).astype(jnp.int32)
        tile_ref[...] = idx[:, None]

        lane = lax.broadcasted_iota(jnp.int32, (BK1, 128), 1)
        oh = (idx[:, None] == lane).astype(jnp.float32)
        r = lax.broadcasted_iota(jnp.int32, (BK1, BK1), 0)
        c = lax.broadcasted_iota(jnp.int32, (BK1, BK1), 1)
        tril = (c < r).astype(jnp.float32)
        csum = lax.dot_general(tril, oh, (((1,), (0,)), ((), ())),
                               preferred_element_type=jnp.float32)
        carry = carry_ref[...]
        rank = jnp.sum((csum + carry) * oh, axis=1)
        oh_ref[pl.ds(i * BK1, BK1), :] = oh
        rank_ref[pl.ds(i * BK1, BK1), :] = rank[:, None]
        carry_ref[...] = carry + jnp.sum(oh, axis=0, keepdims=True)

    @pl.when(i == G1)
    def _():
        counts = carry_ref[...]
        padded = jnp.ceil(counts / BM) * BM
        r2 = lax.broadcasted_iota(jnp.int32, (128, 128), 0)
        c2 = lax.broadcasted_iota(jnp.int32, (128, 128), 1)
        ustrict = (r2 < c2).astype(jnp.float32)
        offs = lax.dot_general(padded, ustrict, (((1,), (0,)), ((), ())),
                               preferred_element_type=jnp.float32)
        eye = (r2 == c2).astype(jnp.float32)
        s_col = lax.dot_general(eye, offs * (1.0 / BM), (((1,), (1,)), ((), ())),
                                preferred_element_type=jnp.float32)
        ind = (c2.astype(jnp.float32) >= s_col).astype(jnp.float32)
        btv = jnp.sum(ind, axis=0, keepdims=True) - 1.0
        btv = jnp.clip(btv, 0.0, float(T - 1))
        lane1 = lax.broadcasted_iota(jnp.int32, (1, 128), 1)
        btv = jnp.where(lane1 == 64, s_col[T, 0], btv)
        bt_ref[...] = btv.astype(jnp.int32)
        seg = lax.dot_general(oh_ref[...], offs, (((1,), (1,)), ((), ())),
                              preferred_element_type=jnp.float32)
        pos_ref[...] = (rank_ref[...] + seg).astype(jnp.int32)


def _k1_call(x, signatures_raw):
    last = G1 - 1
    return pl.pallas_call(
        _k1_body,
        grid=(G1 + 1,),
        in_specs=[
            pl.BlockSpec((BK1, D), lambda i: (jnp.minimum(i, last), 0)),
            pl.BlockSpec((T, D), lambda i: (0, 0)),
        ],
        out_specs=[
            pl.BlockSpec((BK1, 1), lambda i: (jnp.minimum(i, last), 0)),
            pl.BlockSpec((BK1, DP), lambda i: (jnp.minimum(i, last), 0)),
            pl.BlockSpec((B, 1), lambda i: (0, 0)),
            pl.BlockSpec((1, 128), lambda i: (0, 0)),
        ],
        out_shape=[
            jax.ShapeDtypeStruct((B, 1), jnp.int32),
            jax.ShapeDtypeStruct((B, DP), jnp.int32),
            jax.ShapeDtypeStruct((B, 1), jnp.int32),
            jax.ShapeDtypeStruct((1, 128), jnp.int32),
        ],
        scratch_shapes=[
            pltpu.VMEM((1, 128), jnp.float32),
            pltpu.VMEM((B, 128), jnp.float32),
            pltpu.VMEM((B, 1), jnp.float32),
        ],
    )(x, signatures_raw)


@functools.cache
def _sc_kernels():
    mesh = plsc.VectorSubcoreMesh(core_axis_name="c", subcore_axis_name="s")
    nch = TPW // HH
    nbuf = 3

    @functools.partial(
        pl.kernel,
        mesh=mesh,
        out_type=jax.ShapeDtypeStruct((XS, DP), jnp.int32),
        scratch_types=[
            pltpu.VMEM((nch, HH), jnp.int32),
            pltpu.VMEM((nbuf, HH, DP), jnp.int32),
            pltpu.SemaphoreType.DMA,
            pltpu.SemaphoreType.DMA,
        ],
    )
    def k2_scatter(x_hbm, pos_hbm, xs_hbm, pos_v, bufs, in_sem, out_sem):
        wid = lax.axis_index("s") * NC + lax.axis_index("c")
        base = wid * TPW
        pltpu.sync_copy(pos_hbm.at[wid], pos_v)
        ins, outs = {}, {}

        def start_in(h):
            ins[h] = pltpu.async_copy(
                x_hbm.at[pl.ds(base + h * HH, HH)], bufs.at[h % nbuf], in_sem)

        for h in range(nbuf):
            start_in(h)
        for h in range(nch):
            ins[h].wait()
            outs[h] = pltpu.async_copy(
                bufs.at[h % nbuf], xs_hbm.at[pos_v.at[h]], out_sem)
            if h + nbuf < nch:
                outs[h].wait()
                start_in(h + nbuf)
        for h in range(nch):
            if h + nbuf >= nch:
                outs[h].wait()

    @functools.partial(
        pl.kernel,
        mesh=mesh,
        out_type=jax.ShapeDtypeStruct((B, CP), jnp.float32),
        scratch_types=[
            pltpu.VMEM((nch, HH), jnp.int32),
            pltpu.VMEM((nbuf, HH, CP), jnp.float32),
            pltpu.SemaphoreType.DMA,
            pltpu.SemaphoreType.DMA,
        ],
    )
    def k4_gather(ls_hbm, pos_hbm, out_hbm, pos_v, bufs, in_sem, out_sem):
        wid = lax.axis_index("s") * NC + lax.axis_index("c")
        base = wid * TPW
        pltpu.sync_copy(pos_hbm.at[wid], pos_v)
        ins, outs = {}, {}

        def start_in(h):
            ins[h] = pltpu.async_copy(
                ls_hbm.at[pos_v.at[h]], bufs.at[h % nbuf], in_sem)

        for h in range(nbuf):
            start_in(h)
        for h in range(nch):
            ins[h].wait()
            outs[h] = pltpu.async_copy(
                bufs.at[h % nbuf],
                out_hbm.at[pl.ds(base + h * HH, HH)], out_sem)
            if h + nbuf < nch:
                outs[h].wait()
                start_in(h + nbuf)
        for h in range(nch):
            if h + nbuf >= nch:
                outs[h].wait()

    return k2_scatter, k4_gather


def _k3_body(bt_ref, xs_ref, w_ref, b_ref, out_ref, wbf_ref):
    m = pl.program_id(0)

    @pl.when(m == 0)
    def _():
        wbf_ref[pl.ds(C, CP - C)] = jnp.zeros((CP - C, D), jnp.bfloat16)

    @pl.when(m < bt_ref[64])
    def _():
        tprev = bt_ref[jnp.maximum(m - 1, 0)]

        @pl.when(jnp.logical_or(m == 0, bt_ref[m] != tprev))
        def _():
            wbf_ref[pl.ds(0, C)] = w_ref[0].astype(jnp.bfloat16)

        v = xs_ref[...]
        lo = lax.bitcast_convert_type(v << 16, jnp.float32)
        hi = lax.bitcast_convert_type(v & jnp.int32(-65536), jnp.float32)
        xb = jnp.concatenate([lo, hi], axis=1).astype(jnp.bfloat16)
        bb = jnp.pad(b_ref[0], ((0, 0), (0, CP - C)))
        out_ref[...] = lax.dot_general(
            xb, wbf_ref[...], (((1,), (1,)), ((), ())),
            preferred_element_type=jnp.float32) + bb


def _k3_call(bt_flat, xs, W, bp):
    grid_spec = pltpu.PrefetchScalarGridSpec(
        num_scalar_prefetch=1,
        grid=(M,),
        in_specs=[
            pl.BlockSpec((BM, DP), lambda m, bt: (m, 0)),
            pl.BlockSpec((1, C, D), lambda m, bt: (bt[m], 0, 0)),
            pl.BlockSpec((1, 1, C), lambda m, bt: (bt[m], 0, 0)),
        ],
        out_specs=pl.BlockSpec((BM, CP), lambda m, bt: (m, 0)),
        scratch_shapes=[pltpu.VMEM((CP, D), jnp.bfloat16)],
    )
    return pl.pallas_call(
        _k3_body,
        grid_spec=grid_spec,
        out_shape=jax.ShapeDtypeStruct((XS, CP), jnp.float32),
    )(bt_flat, xs, W, bp)


def kernel(x, signatures_raw, W, b):
    k2_scatter, k4_gather = _sc_kernels()
    tile2d, xb, pos2d, bt2d = _k1_call(x, signatures_raw)
    tile_flat = tile2d.reshape(B)
    pos_flat = pos2d.reshape(B)
    bt_flat = bt2d.reshape(128)

    pos4 = pos_flat.reshape(NW, TPW // HH, HH)
    xs = k2_scatter(xb, pos4)

    ls = _k3_call(bt_flat, xs, W, b.reshape(T, 1, C))

    logits_full = k4_gather(ls, pos4)
    return logits_full[:, :C], tile_flat


# speedup vs baseline: 1.2418x; 1.1127x over previous
import functools

import jax
import jax.numpy as jnp
from jax import lax
from jax.experimental import pallas as pl
from jax.experimental.pallas import tpu as pltpu
from jax.experimental.pallas import tpu_sc as plsc

B = 4096
D = 1024
T = 8
C = 1000
CP = 1024
BK1 = 512
G1 = B // BK1
BM = 256
M = B // BM + (T - 1)
XS = M * BM
DP = D // 2
NC = 2
NS = 16
NW = NC * NS
TPW = B // NW
HH = 32


def _k1_body(x_ref, s_ref, tile_ref, xb_ref, pos_ref, bt_ref, carry_ref,
             oh_ref, rank_ref):
    i = pl.program_id(0)

    @pl.when(i == 0)
    def _():
        carry_ref[...] = jnp.zeros_like(carry_ref)

    @pl.when(i < G1)
    def _():
        xv = x_ref[...]
        u = lax.bitcast_convert_type(
            xv.astype(jnp.bfloat16), jnp.uint16).astype(jnp.int32)
        xb_ref[...] = u[:, :DP] | (u[:, DP:] << 16)
        sr = s_ref[...]
        sigs = jnp.where(sr > 0.3, 1.0, jnp.where(sr < -0.3, -1.0, 0.0))
        scores = lax.dot_general(xv, sigs, (((1,), (1,)), ((), ())),
                                 preferred_element_type=jnp.float32)
        idx = jnp.argmax(scores, axis=1).astype(jnp.int32)
        tile_ref[...] = idx[:, None]

        lane = lax.broadcasted_iota(jnp.int32, (BK1, 128), 1)
        oh = (idx[:, None] == lane).astype(jnp.float32)
        r = lax.broadcasted_iota(jnp.int32, (BK1, BK1), 0)
        c = lax.broadcasted_iota(jnp.int32, (BK1, BK1), 1)
        tril = (c < r).astype(jnp.float32)
        csum = lax.dot_general(tril, oh, (((1,), (0,)), ((), ())),
                               preferred_element_type=jnp.float32)
        carry = carry_ref[...]
        rank = jnp.sum((csum + carry) * oh, axis=1)
        oh_ref[pl.ds(i * BK1, BK1), :] = oh
        rank_ref[pl.ds(i * BK1, BK1), :] = rank[:, None]
        carry_ref[...] = carry + jnp.sum(oh, axis=0, keepdims=True)

    @pl.when(i == G1)
    def _():
        counts = carry_ref[...]
        padded = jnp.ceil(counts / BM) * BM
        r2 = lax.broadcasted_iota(jnp.int32, (128, 128), 0)
        c2 = lax.broadcasted_iota(jnp.int32, (128, 128), 1)
        ustrict = (r2 < c2).astype(jnp.float32)
        offs = lax.dot_general(padded, ustrict, (((1,), (0,)), ((), ())),
                               preferred_element_type=jnp.float32)
        eye = (r2 == c2).astype(jnp.float32)
        s_col = lax.dot_general(eye, offs * (1.0 / BM), (((1,), (1,)), ((), ())),
                                preferred_element_type=jnp.float32)
        ind = (c2.astype(jnp.float32) >= s_col).astype(jnp.float32)
        btv = jnp.sum(ind, axis=0, keepdims=True) - 1.0
        btv = jnp.clip(btv, 0.0, float(T - 1))
        lane1 = lax.broadcasted_iota(jnp.int32, (1, 128), 1)
        btv = jnp.where(lane1 == 64, s_col[T, 0], btv)
        bt_ref[...] = btv.astype(jnp.int32)
        seg = lax.dot_general(oh_ref[...], offs, (((1,), (1,)), ((), ())),
                              preferred_element_type=jnp.float32)
        pos_ref[...] = (rank_ref[...] + seg).astype(jnp.int32)


def _k1_call(x, signatures_raw):
    last = G1 - 1
    return pl.pallas_call(
        _k1_body,
        grid=(G1 + 1,),
        in_specs=[
            pl.BlockSpec((BK1, D), lambda i: (jnp.minimum(i, last), 0)),
            pl.BlockSpec((T, D), lambda i: (0, 0)),
        ],
        out_specs=[
            pl.BlockSpec((BK1, 1), lambda i: (jnp.minimum(i, last), 0)),
            pl.BlockSpec((BK1, DP), lambda i: (jnp.minimum(i, last), 0)),
            pl.BlockSpec((B, 1), lambda i: (0, 0)),
            pl.BlockSpec((1, 128), lambda i: (0, 0)),
        ],
        out_shape=[
            jax.ShapeDtypeStruct((B, 1), jnp.int32),
            jax.ShapeDtypeStruct((B, DP), jnp.int32),
            jax.ShapeDtypeStruct((B, 1), jnp.int32),
            jax.ShapeDtypeStruct((1, 128), jnp.int32),
        ],
        scratch_shapes=[
            pltpu.VMEM((1, 128), jnp.float32),
            pltpu.VMEM((B, 128), jnp.float32),
            pltpu.VMEM((B, 1), jnp.float32),
        ],
    )(x, signatures_raw)


@functools.cache
def _sc_kernels():
    mesh = plsc.VectorSubcoreMesh(core_axis_name="c", subcore_axis_name="s")
    nch = TPW // HH
    nbuf = 3

    @functools.partial(
        pl.kernel,
        mesh=mesh,
        out_type=jax.ShapeDtypeStruct((XS, DP), jnp.int32),
        scratch_types=[
            pltpu.VMEM((nch, HH), jnp.int32),
            pltpu.VMEM((nbuf, HH, DP), jnp.int32),
            pltpu.SemaphoreType.DMA,
            pltpu.SemaphoreType.DMA,
        ],
    )
    def k2_scatter(x_hbm, pos_hbm, xs_hbm, pos_v, bufs, in_sem, out_sem):
        wid = lax.axis_index("s") * NC + lax.axis_index("c")
        base = wid * TPW
        pltpu.sync_copy(pos_hbm.at[wid], pos_v)
        ins, outs = {}, {}

        def start_in(h):
            ins[h] = pltpu.async_copy(
                x_hbm.at[pl.ds(base + h * HH, HH)], bufs.at[h % nbuf], in_sem)

        for h in range(nbuf):
            start_in(h)
        for h in range(nch):
            ins[h].wait()
            outs[h] = pltpu.async_copy(
                bufs.at[h % nbuf], xs_hbm.at[pos_v.at[h]], out_sem)
            if h + nbuf < nch:
                outs[h].wait()
                start_in(h + nbuf)
        for h in range(nch):
            if h + nbuf >= nch:
                outs[h].wait()

    @functools.partial(
        pl.kernel,
        mesh=mesh,
        out_type=jax.ShapeDtypeStruct((B, CP), jnp.float32),
        scratch_types=[
            pltpu.VMEM((nch, HH), jnp.int32),
            pltpu.VMEM((nbuf, HH, CP), jnp.float32),
            pltpu.SemaphoreType.DMA,
            pltpu.SemaphoreType.DMA,
        ],
    )
    def k4_gather(ls_hbm, pos_hbm, out_hbm, pos_v, bufs, in_sem, out_sem):
        wid = lax.axis_index("s") * NC + lax.axis_index("c")
        base = wid * TPW
        pltpu.sync_copy(pos_hbm.at[wid], pos_v)
        ins, outs = {}, {}

        def start_in(h):
            ins[h] = pltpu.async_copy(
                ls_hbm.at[pos_v.at[h]], bufs.at[h % nbuf], in_sem)

        for h in range(nbuf):
            start_in(h)
        for h in range(nch):
            ins[h].wait()
            outs[h] = pltpu.async_copy(
                bufs.at[h % nbuf],
                out_hbm.at[pl.ds(base + h * HH, HH)], out_sem)
            if h + nbuf < nch:
                outs[h].wait()
                start_in(h + nbuf)
        for h in range(nch):
            if h + nbuf >= nch:
                outs[h].wait()

    return k2_scatter, k4_gather


def _k3_body(bt_ref, xs_ref, w_ref, b_ref, out_ref, wbf_ref):
    m = pl.program_id(0)

    @pl.when(m == 0)
    def _():
        wbf_ref[pl.ds(C, CP - C)] = jnp.zeros((CP - C, D), jnp.bfloat16)

    @pl.when(m < bt_ref[64])
    def _():
        tprev = bt_ref[jnp.maximum(m - 1, 0)]

        @pl.when(jnp.logical_or(m == 0, bt_ref[m] != tprev))
        def _():
            wbf_ref[pl.ds(0, C)] = w_ref[0].astype(jnp.bfloat16)

        v = xs_ref[...]
        lo = lax.bitcast_convert_type(v << 16, jnp.float32)
        hi = lax.bitcast_convert_type(v & jnp.int32(-65536), jnp.float32)
        xb = jnp.concatenate([lo, hi], axis=1).astype(jnp.bfloat16)
        bb = jnp.pad(b_ref[0], ((0, 0), (0, CP - C)))
        out_ref[...] = lax.dot_general(
            xb, wbf_ref[...], (((1,), (1,)), ((), ())),
            preferred_element_type=jnp.float32) + bb


def _k3_call(bt_flat, xs, W, bp):
    grid_spec = pltpu.PrefetchScalarGridSpec(
        num_scalar_prefetch=1,
        grid=(M,),
        in_specs=[
            pl.BlockSpec((BM, DP), lambda m, bt: (m, 0)),
            pl.BlockSpec((1, C, D), lambda m, bt: (bt[m], 0, 0)),
            pl.BlockSpec((1, 1, C), lambda m, bt: (bt[m], 0, 0)),
        ],
        out_specs=pl.BlockSpec((BM, CP), lambda m, bt: (m, 0)),
        scratch_shapes=[pltpu.VMEM((CP, D), jnp.bfloat16)],
    )
    return pl.pallas_call(
        _k3_body,
        grid_spec=grid_spec,
        out_shape=jax.ShapeDtypeStruct((XS, CP), jnp.float32),
    )(bt_flat, xs, W, bp)


def kernel(x, signatures_raw, W, b):
    k2_scatter, k4_gather = _sc_kernels()
    tile2d, xb, pos2d, bt2d = _k1_call(x, signatures_raw)
    tile_flat = tile2d.reshape(B)
    pos_flat = pos2d.reshape(B)
    bt_flat = bt2d.reshape(128)

    pos4 = pos_flat.reshape(NW, TPW // HH, HH)
    xs = k2_scatter(xb, pos4)

    ls = _k3_call(bt_flat, xs, W, b.reshape(T, 1, C))

    logits_full = k4_gather(ls, pos4)
    return logits_full[:, :C], tile_flat


# speedup vs baseline: 1.2950x; 1.0428x over previous
import functools

import jax
import jax.numpy as jnp
from jax import lax
from jax.experimental import pallas as pl
from jax.experimental.pallas import tpu as pltpu
from jax.experimental.pallas import tpu_sc as plsc

B = 4096
D = 1024
T = 8
C = 1000
CP = 1024
BK1 = 512
G1 = B // BK1
BM = 512
M = B // BM + (T - 1)
XS = M * BM
DP = D // 2
NC = 2
NS = 16
NW = NC * NS
TPW = B // NW
HH = 32


def _k1_body(x_ref, s_ref, tile_ref, xb_ref, pos_ref, bt_ref, carry_ref,
             oh_ref, rank_ref):
    i = pl.program_id(0)

    @pl.when(i == 0)
    def _():
        carry_ref[...] = jnp.zeros_like(carry_ref)

    @pl.when(i < G1)
    def _():
        xv = x_ref[...]
        u = lax.bitcast_convert_type(
            xv.astype(jnp.bfloat16), jnp.uint16).astype(jnp.int32)
        xb_ref[...] = u[:, :DP] | (u[:, DP:] << 16)
        sr = s_ref[...]
        sigs = jnp.where(sr > 0.3, 1.0, jnp.where(sr < -0.3, -1.0, 0.0))
        scores = lax.dot_general(xv, sigs, (((1,), (1,)), ((), ())),
                                 preferred_element_type=jnp.float32)
        idx = jnp.argmax(scores, axis=1).astype(jnp.int32)
        tile_ref[...] = idx[:, None]

        lane = lax.broadcasted_iota(jnp.int32, (BK1, 128), 1)
        oh = (idx[:, None] == lane).astype(jnp.float32)
        r = lax.broadcasted_iota(jnp.int32, (BK1, BK1), 0)
        c = lax.broadcasted_iota(jnp.int32, (BK1, BK1), 1)
        tril = (c < r).astype(jnp.float32)
        csum = lax.dot_general(tril, oh, (((1,), (0,)), ((), ())),
                               preferred_element_type=jnp.float32)
        carry = carry_ref[...]
        rank = jnp.sum((csum + carry) * oh, axis=1)
        oh_ref[pl.ds(i * BK1, BK1), :] = oh
        rank_ref[pl.ds(i * BK1, BK1), :] = rank[:, None]
        carry_ref[...] = carry + jnp.sum(oh, axis=0, keepdims=True)

    @pl.when(i == G1)
    def _():
        counts = carry_ref[...]
        padded = jnp.ceil(counts / BM) * BM
        r2 = lax.broadcasted_iota(jnp.int32, (128, 128), 0)
        c2 = lax.broadcasted_iota(jnp.int32, (128, 128), 1)
        ustrict = (r2 < c2).astype(jnp.float32)
        offs = lax.dot_general(padded, ustrict, (((1,), (0,)), ((), ())),
                               preferred_element_type=jnp.float32)
        eye = (r2 == c2).astype(jnp.float32)
        s_col = lax.dot_general(eye, offs * (1.0 / BM), (((1,), (1,)), ((), ())),
                                preferred_element_type=jnp.float32)
        ind = (c2.astype(jnp.float32) >= s_col).astype(jnp.float32)
        btv = jnp.sum(ind, axis=0, keepdims=True) - 1.0
        btv = jnp.clip(btv, 0.0, float(T - 1))
        lane1 = lax.broadcasted_iota(jnp.int32, (1, 128), 1)
        btv = jnp.where(lane1 == 64, s_col[T, 0], btv)
        bt_ref[...] = btv.astype(jnp.int32)
        seg = lax.dot_general(oh_ref[...], offs, (((1,), (1,)), ((), ())),
                              preferred_element_type=jnp.float32)
        pos_ref[...] = (rank_ref[...] + seg).astype(jnp.int32)


def _k1_call(x, signatures_raw):
    last = G1 - 1
    return pl.pallas_call(
        _k1_body,
        grid=(G1 + 1,),
        in_specs=[
            pl.BlockSpec((BK1, D), lambda i: (jnp.minimum(i, last), 0)),
            pl.BlockSpec((T, D), lambda i: (0, 0)),
        ],
        out_specs=[
            pl.BlockSpec((BK1, 1), lambda i: (jnp.minimum(i, last), 0)),
            pl.BlockSpec((BK1, DP), lambda i: (jnp.minimum(i, last), 0)),
            pl.BlockSpec((B, 1), lambda i: (0, 0)),
            pl.BlockSpec((1, 128), lambda i: (0, 0)),
        ],
        out_shape=[
            jax.ShapeDtypeStruct((B, 1), jnp.int32),
            jax.ShapeDtypeStruct((B, DP), jnp.int32),
            jax.ShapeDtypeStruct((B, 1), jnp.int32),
            jax.ShapeDtypeStruct((1, 128), jnp.int32),
        ],
        scratch_shapes=[
            pltpu.VMEM((1, 128), jnp.float32),
            pltpu.VMEM((B, 128), jnp.float32),
            pltpu.VMEM((B, 1), jnp.float32),
        ],
    )(x, signatures_raw)


@functools.cache
def _sc_kernels():
    mesh = plsc.VectorSubcoreMesh(core_axis_name="c", subcore_axis_name="s")
    nch = TPW // HH
    nbuf = 3

    @functools.partial(
        pl.kernel,
        mesh=mesh,
        out_type=jax.ShapeDtypeStruct((XS, DP), jnp.int32),
        scratch_types=[
            pltpu.VMEM((nch, HH), jnp.int32),
            pltpu.VMEM((nbuf, HH, DP), jnp.int32),
            pltpu.SemaphoreType.DMA,
            pltpu.SemaphoreType.DMA,
        ],
    )
    def k2_scatter(x_hbm, pos_hbm, xs_hbm, pos_v, bufs, in_sem, out_sem):
        wid = lax.axis_index("s") * NC + lax.axis_index("c")
        base = wid * TPW
        pltpu.sync_copy(pos_hbm.at[wid], pos_v)
        ins, outs = {}, {}

        def start_in(h):
            ins[h] = pltpu.async_copy(
                x_hbm.at[pl.ds(base + h * HH, HH)], bufs.at[h % nbuf], in_sem)

        for h in range(nbuf):
            start_in(h)
        for h in range(nch):
            ins[h].wait()
            outs[h] = pltpu.async_copy(
                bufs.at[h % nbuf], xs_hbm.at[pos_v.at[h]], out_sem)
            if h + nbuf < nch:
                outs[h].wait()
                start_in(h + nbuf)
        for h in range(nch):
            if h + nbuf >= nch:
                outs[h].wait()

    @functools.partial(
        pl.kernel,
        mesh=mesh,
        out_type=jax.ShapeDtypeStruct((B, CP), jnp.float32),
        scratch_types=[
            pltpu.VMEM((nch, HH), jnp.int32),
            pltpu.VMEM((nbuf, HH, CP), jnp.float32),
            pltpu.SemaphoreType.DMA,
            pltpu.SemaphoreType.DMA,
        ],
    )
    def k4_gather(ls_hbm, pos_hbm, out_hbm, pos_v, bufs, in_sem, out_sem):
        wid = lax.axis_index("s") * NC + lax.axis_index("c")
        base = wid * TPW
        pltpu.sync_copy(pos_hbm.at[wid], pos_v)
        ins, outs = {}, {}

        def start_in(h):
            ins[h] = pltpu.async_copy(
                ls_hbm.at[pos_v.at[h]], bufs.at[h % nbuf], in_sem)

        for h in range(nbuf):
            start_in(h)
        for h in range(nch):
            ins[h].wait()
            outs[h] = pltpu.async_copy(
                bufs.at[h % nbuf],
                out_hbm.at[pl.ds(base + h * HH, HH)], out_sem)
            if h + nbuf < nch:
                outs[h].wait()
                start_in(h + nbuf)
        for h in range(nch):
            if h + nbuf >= nch:
                outs[h].wait()

    return k2_scatter, k4_gather


def _k3_body(bt_ref, xs_ref, w_ref, b_ref, out_ref, wbf_ref):
    m = pl.program_id(0)

    @pl.when(m == 0)
    def _():
        wbf_ref[pl.ds(C, CP - C)] = jnp.zeros((CP - C, D), jnp.bfloat16)

    @pl.when(m < bt_ref[64])
    def _():
        tprev = bt_ref[jnp.maximum(m - 1, 0)]

        @pl.when(jnp.logical_or(m == 0, bt_ref[m] != tprev))
        def _():
            wbf_ref[pl.ds(0, C)] = w_ref[0].astype(jnp.bfloat16)

        v = xs_ref[...]
        lo = lax.bitcast_convert_type(v << 16, jnp.float32)
        hi = lax.bitcast_convert_type(v & jnp.int32(-65536), jnp.float32)
        xb = jnp.concatenate([lo, hi], axis=1).astype(jnp.bfloat16)
        bb = jnp.pad(b_ref[0], ((0, 0), (0, CP - C)))
        out_ref[...] = lax.dot_general(
            xb, wbf_ref[...], (((1,), (1,)), ((), ())),
            preferred_element_type=jnp.float32) + bb


def _k3_call(bt_flat, xs, W, bp):
    grid_spec = pltpu.PrefetchScalarGridSpec(
        num_scalar_prefetch=1,
        grid=(M,),
        in_specs=[
            pl.BlockSpec((BM, DP), lambda m, bt: (m, 0)),
            pl.BlockSpec((1, C, D), lambda m, bt: (bt[m], 0, 0)),
            pl.BlockSpec((1, 1, C), lambda m, bt: (bt[m], 0, 0)),
        ],
        out_specs=pl.BlockSpec((BM, CP), lambda m, bt: (m, 0)),
        scratch_shapes=[pltpu.VMEM((CP, D), jnp.bfloat16)],
    )
    return pl.pallas_call(
        _k3_body,
        grid_spec=grid_spec,
        out_shape=jax.ShapeDtypeStruct((XS, CP), jnp.float32),
    )(bt_flat, xs, W, bp)


def kernel(x, signatures_raw, W, b):
    k2_scatter, k4_gather = _sc_kernels()
    tile2d, xb, pos2d, bt2d = _k1_call(x, signatures_raw)
    tile_flat = tile2d.reshape(B)
    pos_flat = pos2d.reshape(B)
    bt_flat = bt2d.reshape(128)

    pos4 = pos_flat.reshape(NW, TPW // HH, HH)
    xs = k2_scatter(xb, pos4)

    ls = _k3_call(bt_flat, xs, W, b.reshape(T, 1, C))

    logits_full = k4_gather(ls, pos4)
    return logits_full[:, :C], tile_flat


# speedup vs baseline: 1.3033x; 1.0065x over previous
import functools

import jax
import jax.numpy as jnp
from jax import lax
from jax.experimental import pallas as pl
from jax.experimental.pallas import tpu as pltpu
from jax.experimental.pallas import tpu_sc as plsc

B = 4096
D = 1024
T = 8
C = 1000
CP = 1024
BK1 = 1024
G1 = B // BK1
BM = 512
M = B // BM + (T - 1)
XS = M * BM
DP = D // 2
NC = 2
NS = 16
NW = NC * NS
TPW = B // NW
HH = 32


def _k1_body(x_ref, s_ref, tile_ref, xb_ref, pos_ref, bt_ref, carry_ref,
             oh_ref, rank_ref):
    i = pl.program_id(0)

    @pl.when(i == 0)
    def _():
        carry_ref[...] = jnp.zeros_like(carry_ref)

    @pl.when(i < G1)
    def _():
        xv = x_ref[...]
        u = lax.bitcast_convert_type(
            xv.astype(jnp.bfloat16), jnp.uint16).astype(jnp.int32)
        xb_ref[...] = u[:, :DP] | (u[:, DP:] << 16)
        sr = s_ref[...]
        sigs = jnp.where(sr > 0.3, 1.0, jnp.where(sr < -0.3, -1.0, 0.0))
        scores = lax.dot_general(xv, sigs, (((1,), (1,)), ((), ())),
                                 preferred_element_type=jnp.float32)
        idx = jnp.argmax(scores, axis=1).astype(jnp.int32)
        tile_ref[...] = idx[:, None]

        lane = lax.broadcasted_iota(jnp.int32, (BK1, 128), 1)
        oh = (idx[:, None] == lane).astype(jnp.float32)
        r = lax.broadcasted_iota(jnp.int32, (BK1, BK1), 0)
        c = lax.broadcasted_iota(jnp.int32, (BK1, BK1), 1)
        tril = (c < r).astype(jnp.float32)
        csum = lax.dot_general(tril, oh, (((1,), (0,)), ((), ())),
                               preferred_element_type=jnp.float32)
        carry = carry_ref[...]
        rank = jnp.sum((csum + carry) * oh, axis=1)
        oh_ref[pl.ds(i * BK1, BK1), :] = oh
        rank_ref[pl.ds(i * BK1, BK1), :] = rank[:, None]
        carry_ref[...] = carry + jnp.sum(oh, axis=0, keepdims=True)

    @pl.when(i == G1)
    def _():
        counts = carry_ref[...]
        padded = jnp.ceil(counts / BM) * BM
        r2 = lax.broadcasted_iota(jnp.int32, (128, 128), 0)
        c2 = lax.broadcasted_iota(jnp.int32, (128, 128), 1)
        ustrict = (r2 < c2).astype(jnp.float32)
        offs = lax.dot_general(padded, ustrict, (((1,), (0,)), ((), ())),
                               preferred_element_type=jnp.float32)
        eye = (r2 == c2).astype(jnp.float32)
        s_col = lax.dot_general(eye, offs * (1.0 / BM), (((1,), (1,)), ((), ())),
                                preferred_element_type=jnp.float32)
        ind = (c2.astype(jnp.float32) >= s_col).astype(jnp.float32)
        btv = jnp.sum(ind, axis=0, keepdims=True) - 1.0
        btv = jnp.clip(btv, 0.0, float(T - 1))
        lane1 = lax.broadcasted_iota(jnp.int32, (1, 128), 1)
        btv = jnp.where(lane1 == 64, s_col[T, 0], btv)
        bt_ref[...] = btv.astype(jnp.int32)
        seg = lax.dot_general(oh_ref[...], offs, (((1,), (1,)), ((), ())),
                              preferred_element_type=jnp.float32)
        pos_ref[...] = (rank_ref[...] + seg).astype(jnp.int32)


def _k1_call(x, signatures_raw):
    last = G1 - 1
    return pl.pallas_call(
        _k1_body,
        grid=(G1 + 1,),
        in_specs=[
            pl.BlockSpec((BK1, D), lambda i: (jnp.minimum(i, last), 0)),
            pl.BlockSpec((T, D), lambda i: (0, 0)),
        ],
        out_specs=[
            pl.BlockSpec((BK1, 1), lambda i: (jnp.minimum(i, last), 0)),
            pl.BlockSpec((BK1, DP), lambda i: (jnp.minimum(i, last), 0)),
            pl.BlockSpec((B, 1), lambda i: (0, 0)),
            pl.BlockSpec((1, 128), lambda i: (0, 0)),
        ],
        out_shape=[
            jax.ShapeDtypeStruct((B, 1), jnp.int32),
            jax.ShapeDtypeStruct((B, DP), jnp.int32),
            jax.ShapeDtypeStruct((B, 1), jnp.int32),
            jax.ShapeDtypeStruct((1, 128), jnp.int32),
        ],
        scratch_shapes=[
            pltpu.VMEM((1, 128), jnp.float32),
            pltpu.VMEM((B, 128), jnp.float32),
            pltpu.VMEM((B, 1), jnp.float32),
        ],
    )(x, signatures_raw)


@functools.cache
def _sc_kernels():
    mesh = plsc.VectorSubcoreMesh(core_axis_name="c", subcore_axis_name="s")
    nch = TPW // HH
    nbuf = 3

    @functools.partial(
        pl.kernel,
        mesh=mesh,
        out_type=jax.ShapeDtypeStruct((XS, DP), jnp.int32),
        scratch_types=[
            pltpu.VMEM((nch, HH), jnp.int32),
            pltpu.VMEM((nbuf, HH, DP), jnp.int32),
            pltpu.SemaphoreType.DMA,
            pltpu.SemaphoreType.DMA,
        ],
    )
    def k2_scatter(x_hbm, pos_hbm, xs_hbm, pos_v, bufs, in_sem, out_sem):
        wid = lax.axis_index("s") * NC + lax.axis_index("c")
        base = wid * TPW
        pltpu.sync_copy(pos_hbm.at[wid], pos_v)
        ins, outs = {}, {}

        def start_in(h):
            ins[h] = pltpu.async_copy(
                x_hbm.at[pl.ds(base + h * HH, HH)], bufs.at[h % nbuf], in_sem)

        for h in range(nbuf):
            start_in(h)
        for h in range(nch):
            ins[h].wait()
            outs[h] = pltpu.async_copy(
                bufs.at[h % nbuf], xs_hbm.at[pos_v.at[h]], out_sem)
            if h + nbuf < nch:
                outs[h].wait()
                start_in(h + nbuf)
        for h in range(nch):
            if h + nbuf >= nch:
                outs[h].wait()

    @functools.partial(
        pl.kernel,
        mesh=mesh,
        out_type=jax.ShapeDtypeStruct((B, CP), jnp.float32),
        scratch_types=[
            pltpu.VMEM((nch, HH), jnp.int32),
            pltpu.VMEM((nbuf, HH, CP), jnp.float32),
            pltpu.SemaphoreType.DMA,
            pltpu.SemaphoreType.DMA,
        ],
    )
    def k4_gather(ls_hbm, pos_hbm, out_hbm, pos_v, bufs, in_sem, out_sem):
        wid = lax.axis_index("s") * NC + lax.axis_index("c")
        base = wid * TPW
        pltpu.sync_copy(pos_hbm.at[wid], pos_v)
        ins, outs = {}, {}

        def start_in(h):
            ins[h] = pltpu.async_copy(
                ls_hbm.at[pos_v.at[h]], bufs.at[h % nbuf], in_sem)

        for h in range(nbuf):
            start_in(h)
        for h in range(nch):
            ins[h].wait()
            outs[h] = pltpu.async_copy(
                bufs.at[h % nbuf],
                out_hbm.at[pl.ds(base + h * HH, HH)], out_sem)
            if h + nbuf < nch:
                outs[h].wait()
                start_in(h + nbuf)
        for h in range(nch):
            if h + nbuf >= nch:
                outs[h].wait()

    return k2_scatter, k4_gather


def _k3_body(bt_ref, xs_ref, w_ref, b_ref, out_ref, wbf_ref):
    m = pl.program_id(0)

    @pl.when(m == 0)
    def _():
        wbf_ref[pl.ds(C, CP - C)] = jnp.zeros((CP - C, D), jnp.bfloat16)

    @pl.when(m < bt_ref[64])
    def _():
        tprev = bt_ref[jnp.maximum(m - 1, 0)]

        @pl.when(jnp.logical_or(m == 0, bt_ref[m] != tprev))
        def _():
            wbf_ref[pl.ds(0, C)] = w_ref[0].astype(jnp.bfloat16)

        v = xs_ref[...]
        lo = lax.bitcast_convert_type(v << 16, jnp.float32)
        hi = lax.bitcast_convert_type(v & jnp.int32(-65536), jnp.float32)
        xb = jnp.concatenate([lo, hi], axis=1).astype(jnp.bfloat16)
        bb = jnp.pad(b_ref[0], ((0, 0), (0, CP - C)))
        out_ref[...] = lax.dot_general(
            xb, wbf_ref[...], (((1,), (1,)), ((), ())),
            preferred_element_type=jnp.float32) + bb


def _k3_call(bt_flat, xs, W, bp):
    grid_spec = pltpu.PrefetchScalarGridSpec(
        num_scalar_prefetch=1,
        grid=(M,),
        in_specs=[
            pl.BlockSpec((BM, DP), lambda m, bt: (m, 0)),
            pl.BlockSpec((1, C, D), lambda m, bt: (bt[m], 0, 0)),
            pl.BlockSpec((1, 1, C), lambda m, bt: (bt[m], 0, 0)),
        ],
        out_specs=pl.BlockSpec((BM, CP), lambda m, bt: (m, 0)),
        scratch_shapes=[pltpu.VMEM((CP, D), jnp.bfloat16)],
    )
    return pl.pallas_call(
        _k3_body,
        grid_spec=grid_spec,
        out_shape=jax.ShapeDtypeStruct((XS, CP), jnp.float32),
    )(bt_flat, xs, W, bp)


def kernel(x, signatures_raw, W, b):
    k2_scatter, k4_gather = _sc_kernels()
    tile2d, xb, pos2d, bt2d = _k1_call(x, signatures_raw)
    tile_flat = tile2d.reshape(B)
    pos_flat = pos2d.reshape(B)
    bt_flat = bt2d.reshape(128)

    pos4 = pos_flat.reshape(NW, TPW // HH, HH)
    xs = k2_scatter(xb, pos4)

    ls = _k3_call(bt_flat, xs, W, b.reshape(T, 1, C))

    logits_full = k4_gather(ls, pos4)
    return logits_full[:, :C], tile_flat
